# Initial kernel scaffold; baseline (speedup 1.0000x reference)
import jax
import jax.numpy as jnp
from jax.experimental import pallas as pl


def kernel(user_idx, item_idx, hist_user, hist_item, W_user_anchor, W_user_history, W_item_anchor, W_item_history, W_user_query, W_item_query):
    raise NotImplementedError("write your pallas kernel here")


# speedup vs baseline: 1.1296x; 1.1296x over previous
import functools

import jax
import jax.numpy as jnp
from jax import lax
from jax.experimental import pallas as pl
from jax.experimental.pallas import tpu as pltpu
from jax.experimental.pallas import tpu_sc as plsc

NUM_USERS = 100000
NUM_ITEMS = 1000000
EMBED_DIM = 32
BATCH = 16384
HIST_LEN = 20
USER_PAD = NUM_USERS
ITEM_PAD = NUM_ITEMS

NC = 2
NS = 16
NW = NC * NS
BPW = BATCH // NW
GW = 128
NPW = BPW * HIST_LEN
CH = 2048
NCHUNK = NPW // CH

_MESH = plsc.VectorSubcoreMesh(core_axis_name="c", subcore_axis_name="s")


def _wid():
    return lax.axis_index("s") * NC + lax.axis_index("c")


def _stage1_body(uidx_hbm, iidx_hbm, hu_hbm, hi_hbm, wua_hbm, wia_hbm,
                 uh_out, ih_out, ua_out, ia_out,
                 idx_v, hist_v, anch_v, sem):
    w = _wid()
    nrow = BPW // GW
    base = pl.multiple_of(w * BPW, BPW)
    rbase = pl.multiple_of(w * nrow, nrow)
    for idx_hbm, h_hbm, a_hbm, h_out, a_out in (
            (uidx_hbm, hu_hbm, wua_hbm, uh_out, ua_out),
            (iidx_hbm, hi_hbm, wia_hbm, ih_out, ia_out)):
        pltpu.sync_copy(idx_hbm.at[pl.ds(rbase, nrow)], idx_v)
        cps = []
        for j in range(nrow):
            s = pl.ds(j * GW, GW)
            cps.append(pltpu.async_copy(h_hbm.at[idx_v.at[j]], hist_v.at[s], sem))
            cps.append(pltpu.async_copy(a_hbm.at[idx_v.at[j]], anch_v.at[s], sem))
        for cp in cps:
            cp.wait()
        pltpu.sync_copy(hist_v, h_out.at[pl.ds(base, BPW)])
        pltpu.sync_copy(anch_v, a_out.at[pl.ds(base, BPW)])


def _stage2_body(f_hbm, t_hbm, o_hbm, idx_v, rows_v, sem):
    w = _wid()
    nrow = NPW // GW
    base = pl.multiple_of(w * NPW, NPW)
    rbase = pl.multiple_of(w * nrow, nrow)
    rows_per_chunk = CH // GW
    pltpu.async_copy(f_hbm.at[pl.ds(rbase, nrow)], idx_v, sem).wait()
    for c in range(NCHUNK):
        cps = []
        for j in range(rows_per_chunk):
            cps.append(pltpu.async_copy(
                t_hbm.at[idx_v.at[c * rows_per_chunk + j]],
                rows_v.at[pl.ds(j * GW, GW)], sem))
        for cp in cps:
            cp.wait()
        pltpu.async_copy(rows_v, o_hbm.at[pl.ds(base + c * CH, CH)],
                         sem).wait()


_SC_PARAMS = pltpu.CompilerParams(use_tc_tiling_on_sc=False)

_stage1 = pl.kernel(
    _stage1_body, mesh=_MESH,
    compiler_params=_SC_PARAMS,
    out_type=(
        jax.ShapeDtypeStruct((BATCH, 32), jnp.int32),
        jax.ShapeDtypeStruct((BATCH, 32), jnp.int32),
        jax.ShapeDtypeStruct((BATCH, EMBED_DIM), jnp.float32),
        jax.ShapeDtypeStruct((BATCH, EMBED_DIM), jnp.float32),
    ),
    scratch_types=[
        pltpu.VMEM((BPW // GW, GW), jnp.int32),
        pltpu.VMEM((BPW, 32), jnp.int32),
        pltpu.VMEM((BPW, EMBED_DIM), jnp.float32),
        pltpu.SemaphoreType.DMA,
    ],
)

_stage2 = pl.kernel(
    _stage2_body, mesh=_MESH,
    compiler_params=_SC_PARAMS,
    out_type=jax.ShapeDtypeStruct((BATCH * HIST_LEN, EMBED_DIM), jnp.float32),
    scratch_types=[
        pltpu.VMEM((NPW // GW, GW), jnp.int32),
        pltpu.VMEM((CH, EMBED_DIM), jnp.float32),
        pltpu.SemaphoreType.DMA,
    ],
)


def _mask_body(u_ref, i_ref, mu_ref, mi_ref):
    mu_ref[...] = u_ref[...] != ITEM_PAD
    mi_ref[...] = i_ref[...] != USER_PAD


_MROWS = 2048
_masks = pl.pallas_call(
    _mask_body,
    grid=(BATCH // _MROWS,),
    in_specs=[pl.BlockSpec((_MROWS, HIST_LEN), lambda i: (i, 0))] * 2,
    out_specs=[pl.BlockSpec((_MROWS, HIST_LEN), lambda i: (i, 0))] * 2,
    out_shape=(
        jax.ShapeDtypeStruct((BATCH, HIST_LEN), jnp.bool_),
        jax.ShapeDtypeStruct((BATCH, HIST_LEN), jnp.bool_),
    ),
)


def kernel(user_idx, item_idx, hist_user, hist_item,
           W_user_anchor, W_user_history, W_item_anchor, W_item_history,
           W_user_query, W_item_query):
    user_idx = user_idx.astype(jnp.int32)
    item_idx = item_idx.astype(jnp.int32)
    hist_user = hist_user.astype(jnp.int32)
    hist_item = hist_item.astype(jnp.int32)

    hist_user_p = jnp.pad(hist_user, ((0, 0), (0, 32 - HIST_LEN)))
    hist_item_p = jnp.pad(hist_item, ((0, 0), (0, 32 - HIST_LEN)))

    u_hist32, i_hist32, u_anchor, i_anchor = _stage1(
        user_idx.reshape(BATCH // GW, GW), item_idx.reshape(BATCH // GW, GW),
        hist_user_p, hist_item_p, W_user_anchor, W_item_anchor)
    u_hist = u_hist32[:, :HIST_LEN]
    i_hist = i_hist32[:, :HIST_LEN]

    u_rows = _stage2(u_hist.reshape(BATCH * HIST_LEN // GW, GW), W_item_history)
    i_rows = _stage2(i_hist.reshape(BATCH * HIST_LEN // GW, GW), W_user_history)

    u_mask, i_mask = _masks(u_hist, i_hist)

    return (u_anchor, u_rows.reshape(BATCH, HIST_LEN, EMBED_DIM), W_user_query,
            i_anchor, i_rows.reshape(BATCH, HIST_LEN, EMBED_DIM), W_item_query,
            u_mask, i_mask)


# speedup vs baseline: 1.1350x; 1.0048x over previous
import functools

import jax
import jax.numpy as jnp
from jax import lax
from jax.experimental import pallas as pl
from jax.experimental.pallas import tpu as pltpu
from jax.experimental.pallas import tpu_sc as plsc

NUM_USERS = 100000
NUM_ITEMS = 1000000
EMBED_DIM = 32
BATCH = 16384
HIST_LEN = 20
USER_PAD = NUM_USERS
ITEM_PAD = NUM_ITEMS

NC = 2
NS = 16
NW = NC * NS
BPW = BATCH // NW
GW = 128
NPW = BPW * HIST_LEN
CH = 2048
NCHUNK = NPW // CH

_MESH = plsc.VectorSubcoreMesh(core_axis_name="c", subcore_axis_name="s")


def _wid():
    return lax.axis_index("s") * NC + lax.axis_index("c")


def _stage1_body(uidx_hbm, iidx_hbm, hu_hbm, hi_hbm, wua_hbm, wia_hbm,
                 uf_out, if_out, ua_out, ia_out,
                 idx_v, hist_v, anch_v, flat_v, sem):
    w = _wid()
    nrow = BPW // GW
    base = pl.multiple_of(w * BPW, BPW)
    rbase = pl.multiple_of(w * nrow, nrow)
    fbase = pl.multiple_of(w * NPW, NPW)
    lane = jnp.arange(16, dtype=jnp.int32)
    for idx_hbm, h_hbm, a_hbm, f_out, a_out in (
            (uidx_hbm, hu_hbm, wua_hbm, uf_out, ua_out),
            (iidx_hbm, hi_hbm, wia_hbm, if_out, ia_out)):
        pltpu.sync_copy(idx_hbm.at[pl.ds(rbase, nrow)], idx_v)
        cps = []
        for j in range(nrow):
            s = pl.ds(j * GW, GW)
            cps.append(pltpu.async_copy(h_hbm.at[idx_v.at[j]], hist_v.at[s], sem))
            cps.append(pltpu.async_copy(a_hbm.at[idx_v.at[j]], anch_v.at[s], sem))
        for cp in cps:
            cp.wait()
        @pl.loop(0, BPW // 4)
        def _compact(g):
            for k in range(5):
                p0 = 16 * k
                r0, r1 = p0 // HIST_LEN, (p0 + 15) // HIST_LEN
                if r0 == r1:
                    rows = lane * 0 + r0
                else:
                    rows = jnp.where(lane < r1 * HIST_LEN - p0, r0, r1)
                cols = lane + p0 - rows * HIST_LEN
                x = plsc.load_gather(hist_v, [g * 4 + rows, cols])
                flat_v[pl.ds(pl.multiple_of(g * 80 + k * 16, 16), 16)] = x

        pltpu.sync_copy(flat_v, f_out.at[pl.ds(fbase, NPW)])
        pltpu.sync_copy(anch_v, a_out.at[pl.ds(base, BPW)])


def _stage2_body(f_hbm, t_hbm, o_hbm, idx_v, rows_v, sem):
    w = _wid()
    nrow = NPW // GW
    base = pl.multiple_of(w * NPW, NPW)
    rbase = pl.multiple_of(w * nrow, nrow)
    rows_per_chunk = CH // GW
    pltpu.async_copy(f_hbm.at[pl.ds(rbase, nrow)], idx_v, sem).wait()
    for c in range(NCHUNK):
        cps = []
        for j in range(rows_per_chunk):
            cps.append(pltpu.async_copy(
                t_hbm.at[idx_v.at[c * rows_per_chunk + j]],
                rows_v.at[pl.ds(j * GW, GW)], sem))
        for cp in cps:
            cp.wait()
        pltpu.async_copy(rows_v, o_hbm.at[pl.ds(base + c * CH, CH)],
                         sem).wait()


_SC_PARAMS = pltpu.CompilerParams(use_tc_tiling_on_sc=False)
_SC_PARAMS_NLP = pltpu.CompilerParams(use_tc_tiling_on_sc=False,
                                      needs_layout_passes=False)

_stage1 = pl.kernel(
    _stage1_body, mesh=_MESH,
    compiler_params=_SC_PARAMS_NLP,
    out_type=(
        jax.ShapeDtypeStruct((BATCH * HIST_LEN,), jnp.int32),
        jax.ShapeDtypeStruct((BATCH * HIST_LEN,), jnp.int32),
        jax.ShapeDtypeStruct((BATCH, EMBED_DIM), jnp.float32),
        jax.ShapeDtypeStruct((BATCH, EMBED_DIM), jnp.float32),
    ),
    scratch_types=[
        pltpu.VMEM((BPW // GW, GW), jnp.int32),
        pltpu.VMEM((BPW, 32), jnp.int32),
        pltpu.VMEM((BPW, EMBED_DIM), jnp.float32),
        pltpu.VMEM((NPW,), jnp.int32),
        pltpu.SemaphoreType.DMA,
    ],
)

_stage2 = pl.kernel(
    _stage2_body, mesh=_MESH,
    compiler_params=_SC_PARAMS,
    out_type=jax.ShapeDtypeStruct((BATCH * HIST_LEN, EMBED_DIM), jnp.float32),
    scratch_types=[
        pltpu.VMEM((NPW // GW, GW), jnp.int32),
        pltpu.VMEM((CH, EMBED_DIM), jnp.float32),
        pltpu.SemaphoreType.DMA,
    ],
)


def _mask_body(u_ref, i_ref, mu_ref, mi_ref):
    mu_ref[...] = u_ref[...] != ITEM_PAD
    mi_ref[...] = i_ref[...] != USER_PAD


_MROWS = 4096
_masks = pl.pallas_call(
    _mask_body,
    grid=(BATCH // _MROWS,),
    in_specs=[pl.BlockSpec((_MROWS, HIST_LEN), lambda i: (i, 0))] * 2,
    out_specs=[pl.BlockSpec((_MROWS, HIST_LEN), lambda i: (i, 0))] * 2,
    out_shape=(
        jax.ShapeDtypeStruct((BATCH, HIST_LEN), jnp.bool_),
        jax.ShapeDtypeStruct((BATCH, HIST_LEN), jnp.bool_),
    ),
)


def kernel(user_idx, item_idx, hist_user, hist_item,
           W_user_anchor, W_user_history, W_item_anchor, W_item_history,
           W_user_query, W_item_query):
    user_idx = user_idx.astype(jnp.int32)
    item_idx = item_idx.astype(jnp.int32)
    hist_user = hist_user.astype(jnp.int32)
    hist_item = hist_item.astype(jnp.int32)

    hist_user_p = jnp.pad(hist_user, ((0, 0), (0, 32 - HIST_LEN)))
    hist_item_p = jnp.pad(hist_item, ((0, 0), (0, 32 - HIST_LEN)))

    u_flat, i_flat, u_anchor, i_anchor = _stage1(
        user_idx.reshape(BATCH // GW, GW), item_idx.reshape(BATCH // GW, GW),
        hist_user_p, hist_item_p, W_user_anchor, W_item_anchor)

    u_rows = _stage2(u_flat.reshape(BATCH * HIST_LEN // GW, GW), W_item_history)
    i_rows = _stage2(i_flat.reshape(BATCH * HIST_LEN // GW, GW), W_user_history)

    u_mask, i_mask = _masks(u_flat.reshape(BATCH, HIST_LEN),
                            i_flat.reshape(BATCH, HIST_LEN))

    return (u_anchor, u_rows.reshape(BATCH, HIST_LEN, EMBED_DIM), W_user_query,
            i_anchor, i_rows.reshape(BATCH, HIST_LEN, EMBED_DIM), W_item_query,
            u_mask, i_mask)


# speedup vs baseline: 1.2937x; 1.1398x over previous
import functools

import jax
import jax.numpy as jnp
from jax import lax
from jax.experimental import pallas as pl
from jax.experimental.pallas import tpu as pltpu
from jax.experimental.pallas import tpu_sc as plsc

NUM_USERS = 100000
NUM_ITEMS = 1000000
EMBED_DIM = 32
BATCH = 16384
HIST_LEN = 20
USER_PAD = NUM_USERS
ITEM_PAD = NUM_ITEMS

NC = 2
NS = 16
NW = NC * NS
BPW = BATCH // NW
GW = 128
NPW = BPW * HIST_LEN
CH = 2048
NCHUNK = NPW // CH

_MESH = plsc.VectorSubcoreMesh(core_axis_name="c", subcore_axis_name="s")


def _wid():
    return lax.axis_index("s") * NC + lax.axis_index("c")


def _stage1_body(uidx_hbm, iidx_hbm, hu_hbm, hi_hbm, wua_hbm, wia_hbm,
                 uf_out, if_out, ua_out, ia_out,
                 idx_v, idx4_v, hist_v, anch_v, flat_v, sem):
    w = _wid()
    nrow = BPW // GW
    base = pl.multiple_of(w * BPW, BPW)
    rbase = pl.multiple_of(w * nrow, nrow)
    fbase = pl.multiple_of(w * NPW, NPW)
    lane = jnp.arange(16, dtype=jnp.int32)
    for idx_hbm, h_hbm, a_hbm, f_out, a_out in (
            (uidx_hbm, hu_hbm, wua_hbm, uf_out, ua_out),
            (iidx_hbm, hi_hbm, wia_hbm, if_out, ia_out)):
        pltpu.sync_copy(idx_hbm.at[pl.ds(rbase, nrow)], idx_v)
        for j in range(nrow):
            jv = lane * 0 + j
            for m in range(GW // 16):
                cv = m * 16 + lane
                v = plsc.load_gather(idx_v, [jv, cv])
                plsc.store_scatter(idx4_v, [jv, cv],
                                   jax.lax.shift_right_logical(v, 2))
        cps = []
        for j in range(nrow):
            s = pl.ds(j * GW, GW)
            cps.append(pltpu.async_copy(h_hbm.at[idx4_v.at[j]], hist_v.at[s], sem))
            cps.append(pltpu.async_copy(a_hbm.at[idx_v.at[j]], anch_v.at[s], sem))
        for cp in cps:
            cp.wait()
        @pl.loop(0, BPW // 4)
        def _compact(g):
            for k in range(5):
                p0 = 16 * k
                r0, r1 = p0 // HIST_LEN, (p0 + 15) // HIST_LEN
                if r0 == r1:
                    rows = lane * 0 + r0
                else:
                    rows = jnp.where(lane < r1 * HIST_LEN - p0, r0, r1)
                cols = lane + p0 - rows * HIST_LEN
                a = g * 4 + rows
                u = plsc.load_gather(
                    idx_v, [jax.lax.shift_right_logical(a, 7), a & 127])
                x = plsc.load_gather(hist_v, [a, (u & 3) * HIST_LEN + cols])
                flat_v[pl.ds(pl.multiple_of(g * 80 + k * 16, 16), 16)] = x

        pltpu.sync_copy(flat_v, f_out.at[pl.ds(fbase, NPW)])
        pltpu.sync_copy(anch_v, a_out.at[pl.ds(base, BPW)])


def _stage2_body(f_hbm, t_hbm, o_hbm, idx_v, rows_v, sem):
    w = _wid()
    nrow = NPW // GW
    base = pl.multiple_of(w * NPW, NPW)
    rbase = pl.multiple_of(w * nrow, nrow)
    rows_per_chunk = CH // GW
    pltpu.async_copy(f_hbm.at[pl.ds(rbase, nrow)], idx_v, sem).wait()
    for c in range(NCHUNK):
        cps = []
        for j in range(rows_per_chunk):
            cps.append(pltpu.async_copy(
                t_hbm.at[idx_v.at[c * rows_per_chunk + j]],
                rows_v.at[pl.ds(j * GW, GW)], sem))
        for cp in cps:
            cp.wait()
        pltpu.async_copy(rows_v, o_hbm.at[pl.ds(base + c * CH, CH)],
                         sem).wait()


_SC_PARAMS = pltpu.CompilerParams(use_tc_tiling_on_sc=False)
_SC_PARAMS_NLP = pltpu.CompilerParams(use_tc_tiling_on_sc=False,
                                      needs_layout_passes=False)

_stage1 = pl.kernel(
    _stage1_body, mesh=_MESH,
    compiler_params=_SC_PARAMS_NLP,
    out_type=(
        jax.ShapeDtypeStruct((BATCH * HIST_LEN,), jnp.int32),
        jax.ShapeDtypeStruct((BATCH * HIST_LEN,), jnp.int32),
        jax.ShapeDtypeStruct((BATCH, EMBED_DIM), jnp.float32),
        jax.ShapeDtypeStruct((BATCH, EMBED_DIM), jnp.float32),
    ),
    scratch_types=[
        pltpu.VMEM((BPW // GW, GW), jnp.int32),
        pltpu.VMEM((BPW // GW, GW), jnp.int32),
        pltpu.VMEM((BPW, 4 * HIST_LEN), jnp.int32),
        pltpu.VMEM((BPW, EMBED_DIM), jnp.float32),
        pltpu.VMEM((NPW,), jnp.int32),
        pltpu.SemaphoreType.DMA,
    ],
)

_stage2 = pl.kernel(
    _stage2_body, mesh=_MESH,
    compiler_params=_SC_PARAMS,
    out_type=jax.ShapeDtypeStruct((BATCH * HIST_LEN, EMBED_DIM), jnp.float32),
    scratch_types=[
        pltpu.VMEM((NPW // GW, GW), jnp.int32),
        pltpu.VMEM((CH, EMBED_DIM), jnp.float32),
        pltpu.SemaphoreType.DMA,
    ],
)


def _mask_body(u_ref, i_ref, mu_ref, mi_ref):
    mu_ref[...] = u_ref[...] != ITEM_PAD
    mi_ref[...] = i_ref[...] != USER_PAD


_MROWS = 4096
_masks = pl.pallas_call(
    _mask_body,
    grid=(BATCH // _MROWS,),
    in_specs=[pl.BlockSpec((_MROWS, HIST_LEN), lambda i: (i, 0))] * 2,
    out_specs=[pl.BlockSpec((_MROWS, HIST_LEN), lambda i: (i, 0))] * 2,
    out_shape=(
        jax.ShapeDtypeStruct((BATCH, HIST_LEN), jnp.bool_),
        jax.ShapeDtypeStruct((BATCH, HIST_LEN), jnp.bool_),
    ),
)


def kernel(user_idx, item_idx, hist_user, hist_item,
           W_user_anchor, W_user_history, W_item_anchor, W_item_history,
           W_user_query, W_item_query):
    user_idx = user_idx.astype(jnp.int32)
    item_idx = item_idx.astype(jnp.int32)
    hist_user = hist_user.astype(jnp.int32)
    hist_item = hist_item.astype(jnp.int32)

    hist_user_q = hist_user.reshape(NUM_USERS // 4, 4 * HIST_LEN)
    hist_item_q = hist_item.reshape(NUM_ITEMS // 4, 4 * HIST_LEN)

    u_flat, i_flat, u_anchor, i_anchor = _stage1(
        user_idx.reshape(BATCH // GW, GW), item_idx.reshape(BATCH // GW, GW),
        hist_user_q, hist_item_q, W_user_anchor, W_item_anchor)

    u_rows = _stage2(u_flat.reshape(BATCH * HIST_LEN // GW, GW), W_item_history)
    i_rows = _stage2(i_flat.reshape(BATCH * HIST_LEN // GW, GW), W_user_history)

    u_mask, i_mask = _masks(u_flat.reshape(BATCH, HIST_LEN),
                            i_flat.reshape(BATCH, HIST_LEN))

    return (u_anchor, u_rows.reshape(BATCH, HIST_LEN, EMBED_DIM), W_user_query,
            i_anchor, i_rows.reshape(BATCH, HIST_LEN, EMBED_DIM), W_item_query,
            u_mask, i_mask)
